# baseline (speedup 1.0000x reference)
import dataclasses
import functools

import jax
import jax.numpy as jnp
import numpy as np
from jax import lax
from jax.experimental import pallas as pl
from jax.experimental.pallas import tpu as pltpu
from jax.experimental.pallas import tpu_sc as plsc

N = 1_000_000
TABLE_T = 2_097_152
NCH = 15
NPAD = 1 << 20
NW = 32
PER_W = NPAD // NW
CHUNK = 16384
NCHUNKS = PER_W // CHUNK
NBUF = 3
NSTR = 8
SUB = CHUNK // NSTR

P2 = np.int32(np.uint32(2654435761).view(np.int32))
P3 = np.int32(805459861)
MASK = np.int32(TABLE_T - 1)


def _sc_gather(tab_flat, cflat):
    mesh = plsc.VectorSubcoreMesh(core_axis_name="c", subcore_axis_name="s")
    cp = pltpu.CompilerParams()
    if "needs_layout_passes" in pltpu.CompilerParams.__dataclass_fields__:
        cp = dataclasses.replace(cp, needs_layout_passes=False)

    @functools.partial(
        pl.kernel,
        mesh=mesh,
        compiler_params=cp,
        out_type=(
            jax.ShapeDtypeStruct((NCH * NPAD,), jnp.float32),
            jax.ShapeDtypeStruct((NW * 2 * 16,), jnp.float32),
        ),
        scratch_types=[
            pltpu.VMEM((3 * CHUNK,), jnp.int32),
            pltpu.VMEM((CHUNK,), jnp.int32),
            pltpu.VMEM((CHUNK,), jnp.float32),
            pltpu.VMEM((CHUNK,), jnp.float32),
            pltpu.VMEM((16,), jnp.float32),
            pltpu.VMEM((16,), jnp.float32),
            *([pltpu.SemaphoreType.DMA] * (2 * NSTR)),
            pltpu.SemaphoreType.DMA,
            pltpu.SemaphoreType.DMA,
        ],
    )
    def k(tab_hbm, cf_hbm, g_hbm, part_hbm, c3, idx, gb0, gb1, acc,
          accsq, *sems):
        semg = (sems[:NSTR], sems[NSTR:2 * NSTR])
        semw = sems[2 * NSTR:]
        wid = lax.axis_index("s") * 2 + lax.axis_index("c")
        base_w = wid * PER_W
        acc[...] = jnp.zeros((16,), jnp.float32)
        accsq[...] = jnp.zeros((16,), jnp.float32)
        gbuf = (gb0, gb1)
        lane = lax.iota(jnp.int32, 16)

        @pl.loop(0, NCHUNKS)
        def _chunk(ci):
            base = pl.multiple_of(base_w + ci * CHUNK, CHUNK)
            pltpu.sync_copy(cf_hbm.at[pl.ds(base * 3, 3 * CHUNK)], c3)

            @pl.loop(0, CHUNK, step=16)
            def _hash16(j):
                i0 = j * 3 + lane * 3
                x = plsc.load_gather(c3, [i0])
                y = plsc.load_gather(c3, [i0 + 1])
                z = plsc.load_gather(c3, [i0 + 2])
                h = x ^ (y * P2) ^ (z * P3)
                idx[pl.ds(j, 16)] = h & MASK

            def src(ch, s):
                off = pl.multiple_of(np.int32(ch) * TABLE_T, TABLE_T)
                sub_idx = idx.at[pl.ds(s * SUB, SUB)]
                return tab_hbm.at[pl.ds(off, TABLE_T)].at[sub_idx]

            def wdst(ch):
                dst0 = pl.multiple_of(np.int32(ch * NPAD) + base, CHUNK)
                return g_hbm.at[pl.ds(dst0, CHUNK)]

            def fire(ch):
                b = ch % 2
                if ch >= 2:
                    pltpu.make_async_copy(gbuf[b], wdst(ch - 2), semw[b]).wait()
                for s in range(NSTR):
                    pltpu.async_copy(
                        src(ch, s), gbuf[b].at[pl.ds(s * SUB, SUB)], semg[b][s]
                    )

            def wait_gather(ch):
                b = ch % 2
                for s in range(NSTR):
                    pltpu.make_async_copy(
                        src(ch, s), gbuf[b].at[pl.ds(s * SUB, SUB)], semg[b][s]
                    ).wait()

            fire(0)
            for ch in range(NCH):
                b = ch % 2
                if ch + 1 < NCH:
                    fire(ch + 1)
                wait_gather(ch)
                if ch < 3:
                    gb = gbuf[b]

                    @pl.loop(0, CHUNK, step=16)
                    def _st16(j):
                        xv = gb[pl.ds(j, 16)]
                        m = jnp.where(base + j < N,
                                      jnp.float32(1.0), jnp.float32(0.0))
                        acc[...] += xv * m
                        accsq[...] += xv * xv * m

                pltpu.async_copy(gbuf[b], wdst(ch), semw[b])
            pltpu.make_async_copy(gbuf[1], wdst(NCH - 2), semw[1]).wait()
            pltpu.make_async_copy(gbuf[0], wdst(NCH - 1), semw[0]).wait()

        pltpu.sync_copy(acc, part_hbm.at[pl.ds(pl.multiple_of(wid * 32, 16), 16)])
        pltpu.sync_copy(
            accsq, part_hbm.at[pl.ds(pl.multiple_of(wid * 32 + 16, 16), 16)]
        )

    return k(tab_flat, cflat)


def _tc_act_body(g_ref, p_ref, fac_ref, o_ref):
    p = p_ref[...]
    fl = lax.broadcasted_iota(jnp.int32, p.shape, 0) * 128 + lax.broadcasted_iota(
        jnp.int32, p.shape, 1
    )
    is_sum = (fl % 32) < 16
    S = jnp.sum(jnp.where(is_sum, p, 0.0))
    SS = jnp.sum(jnp.where(is_sum, 0.0, p))
    M = jnp.float32(3 * N)
    mu = S / M
    var = (SS - S * S / M) / (M - 1.0)
    inv_sd = lax.rsqrt(var)
    f = fac_ref[0, 0]
    vs = fac_ref[0, 1]
    s1 = 2.0 * f / vs
    g = g_ref[...]
    rows = lax.broadcasted_iota(jnp.int32, g.shape, 0)
    sig = 1.0 / (1.0 + jnp.exp(-g))
    sig4 = 1.0 / (1.0 + jnp.exp(-(g - 4.0)))
    dm = (g - mu) * inv_sd * (s1 / 6.0)
    o_ref[...] = jnp.where(
        rows < 3,
        dm,
        jnp.where(
            rows < 7,
            g,
            jnp.where(rows < 10, sig * s1, jnp.where(rows == 13, sig4, sig)),
        ),
    )


def _tc_activate(g_raw, partials, fac):
    BN = 8192
    grid = (pl.cdiv(N, BN),)
    return pl.pallas_call(
        _tc_act_body,
        grid=grid,
        in_specs=[
            pl.BlockSpec((NCH, BN), lambda i: (0, i)),
            pl.BlockSpec((8, 128), lambda i: (0, 0)),
            pl.BlockSpec((1, 2), lambda i: (0, 0)),
        ],
        out_specs=pl.BlockSpec((NCH, BN), lambda i: (0, i)),
        out_shape=jax.ShapeDtypeStruct((NCH, N), jnp.float32),
    )(g_raw, partials, fac)


def kernel(hash_table, coordinates, far, voxel_size):
    cflat = jnp.pad(coordinates, ((0, NPAD - N), (0, 0))).reshape(-1)
    g_raw, partials = _sc_gather(hash_table.reshape(-1), cflat)
    fac = jnp.stack(
        [far[0].astype(jnp.float32),
         jnp.asarray(voxel_size, jnp.float32)]
    ).reshape(1, 2)
    return _tc_activate(
        g_raw.reshape(NCH, NPAD), partials.reshape(8, 128), fac
    )


# speedup vs baseline: 1.4158x; 1.4158x over previous
import dataclasses
import functools

import jax
import jax.numpy as jnp
import numpy as np
from jax import lax
from jax.experimental import pallas as pl
from jax.experimental.pallas import tpu as pltpu
from jax.experimental.pallas import tpu_sc as plsc

N = 1_000_000
TABLE_T = 2_097_152
NCH = 15
NPAD = 1 << 20
NW = 32
PER_W = NPAD // NW

TK = 2048
T_PER_W = TABLE_T // NW
TCHUNKS = T_PER_W // TK

CG = 2048
GCHUNKS = PER_W // CG

P2 = np.int32(np.uint32(2654435761).view(np.int32))
P3 = np.int32(805459861)
MASK = np.int32(TABLE_T - 1)


def _cparams():
    cp = pltpu.CompilerParams()
    fields = pltpu.CompilerParams.__dataclass_fields__
    if "needs_layout_passes" in fields:
        cp = dataclasses.replace(cp, needs_layout_passes=False)
    if "use_tc_tiling_on_sc" in fields:
        cp = dataclasses.replace(cp, use_tc_tiling_on_sc=False)
    return cp


def _sc_transpose(tab_flat):
    mesh = plsc.VectorSubcoreMesh(core_axis_name="c", subcore_axis_name="s")

    @functools.partial(
        pl.kernel,
        mesh=mesh,
        compiler_params=_cparams(),
        out_type=jax.ShapeDtypeStruct((TABLE_T * 16,), jnp.float32),
        scratch_types=[
            pltpu.VMEM((16 * TK,), jnp.float32),
            pltpu.VMEM((16 * TK,), jnp.float32),
            pltpu.SemaphoreType.DMA,
        ],
    )
    def k(tab_hbm, t16_hbm, cbuf, obuf, sem):
        wid = lax.axis_index("s") * 2 + lax.axis_index("c")
        t_base_w = wid * T_PER_W
        lane = lax.iota(jnp.int32, 16)
        lK = jnp.minimum(lane, 14) * TK

        @pl.loop(0, TCHUNKS)
        def _chunk(ci):
            t0 = pl.multiple_of(t_base_w + ci * TK, TK)
            for ch in range(NCH):
                pltpu.async_copy(
                    tab_hbm.at[pl.ds(pl.multiple_of(ch * TABLE_T, TK) + t0, TK)],
                    cbuf.at[pl.ds(ch * TK, TK)],
                    sem,
                )
            for ch in range(NCH):
                pltpu.make_async_copy(
                    tab_hbm.at[pl.ds(pl.multiple_of(ch * TABLE_T, TK) + t0, TK)],
                    cbuf.at[pl.ds(ch * TK, TK)],
                    sem,
                ).wait()

            @pl.loop(0, TK)
            def _row(r):
                v = plsc.load_gather(cbuf, [lK + r])
                obuf[pl.ds(r * 16, 16)] = v

            pltpu.sync_copy(obuf, t16_hbm.at[pl.ds(t0 * 16, 16 * TK)])

    return k(tab_flat)


def _sc_gather(t16, cflat):
    mesh = plsc.VectorSubcoreMesh(core_axis_name="c", subcore_axis_name="s")

    @functools.partial(
        pl.kernel,
        mesh=mesh,
        compiler_params=_cparams(),
        out_type=(
            jax.ShapeDtypeStruct((NCH * NPAD,), jnp.float32),
            jax.ShapeDtypeStruct((NW * 2 * 16,), jnp.float32),
        ),
        scratch_types=[
            pltpu.VMEM((3 * CG,), jnp.int32),
            pltpu.VMEM((CG,), jnp.int32),
            pltpu.VMEM((CG,), jnp.int32),
            pltpu.VMEM((CG, 16), jnp.float32),
            pltpu.VMEM((CG, 16), jnp.float32),
            pltpu.VMEM((16 * CG,), jnp.float32),
            pltpu.VMEM((16,), jnp.float32),
            pltpu.VMEM((16,), jnp.float32),
            pltpu.SemaphoreType.DMA,
            pltpu.SemaphoreType.DMA,
            pltpu.SemaphoreType.DMA,
        ],
    )
    def k(t16_hbm, cf_hbm, g_hbm, part_hbm, c3, idx0, idx1, gb0, gb1, gbx,
          acc, accsq, sg0, sg1, semw):
        wid = lax.axis_index("s") * 2 + lax.axis_index("c")
        base_w = wid * PER_W
        acc[...] = jnp.zeros((16,), jnp.float32)
        accsq[...] = jnp.zeros((16,), jnp.float32)
        idxs = (idx0, idx1)
        gbs = (gb0, gb1)
        sgs = (sg0, sg1)
        lane = lax.iota(jnp.int32, 16)
        laneCG = lane * CG

        def chunk_base(k_):
            return pl.multiple_of(base_w + k_ * CG, CG)

        def hash_chunk(k_, b):
            base = chunk_base(k_)
            idx = idxs[b]
            pltpu.sync_copy(cf_hbm.at[pl.ds(base * 3, 3 * CG)], c3)

            @pl.loop(0, CG, step=16)
            def _hash16(j):
                i0 = j * 3 + lane * 3
                x = plsc.load_gather(c3, [i0])
                y = plsc.load_gather(c3, [i0 + 1])
                z = plsc.load_gather(c3, [i0 + 2])
                h = x ^ (y * P2) ^ (z * P3)
                idx[pl.ds(j, 16)] = h & MASK

        def fire(b):
            pltpu.async_copy(t16_hbm.at[idxs[b]], gbs[b], sgs[b])

        def wait_gather(b):
            pltpu.make_async_copy(t16_hbm.at[idxs[b]], gbs[b], sgs[b]).wait()

        def transform(k_, b):
            base = chunk_base(k_)
            gb = gbs[b]

            @pl.loop(0, CG)
            def _row(r):
                v = gb[r, :]
                m = jnp.where(base + r < N, jnp.float32(1.0), jnp.float32(0.0))
                acc[...] += v * m
                accsq[...] += v * v * m
                plsc.store_scatter(gbx, [laneCG + r], v)

        def fire_writes(k_):
            base = chunk_base(k_)
            for ch in range(NCH):
                dst0 = pl.multiple_of(np.int32(ch * NPAD) + base, CG)
                pltpu.async_copy(
                    gbx.at[pl.ds(ch * CG, CG)],
                    g_hbm.at[pl.ds(dst0, CG)],
                    semw,
                )

        def wait_writes(k_):
            base = chunk_base(k_)
            for ch in range(NCH):
                dst0 = pl.multiple_of(np.int32(ch * NPAD) + base, CG)
                pltpu.make_async_copy(
                    gbx.at[pl.ds(ch * CG, CG)],
                    g_hbm.at[pl.ds(dst0, CG)],
                    semw,
                ).wait()

        hash_chunk(0, 0)
        fire(0)

        @pl.loop(0, GCHUNKS // 2)
        def _pair(kk):
            k0 = kk * 2
            hash_chunk(k0 + 1, 1)
            fire(1)
            wait_gather(0)
            transform(k0, 0)
            fire_writes(k0)

            @pl.when(k0 + 2 < GCHUNKS)
            def _():
                hash_chunk(k0 + 2, 0)
                fire(0)

            wait_gather(1)
            wait_writes(k0)
            transform(k0 + 1, 1)
            fire_writes(k0 + 1)
            wait_writes(k0 + 1)

        pltpu.sync_copy(acc, part_hbm.at[pl.ds(pl.multiple_of(wid * 32, 16), 16)])
        pltpu.sync_copy(
            accsq, part_hbm.at[pl.ds(pl.multiple_of(wid * 32 + 16, 16), 16)]
        )

    return k(t16, cflat)


def _tc_act_body(g_ref, p_ref, fac_ref, o_ref):
    p = p_ref[...]
    fl = lax.broadcasted_iota(jnp.int32, p.shape, 0) * 128 + lax.broadcasted_iota(
        jnp.int32, p.shape, 1
    )
    ch3 = (fl % 16) < 3
    is_sum = ((fl % 32) < 16) & ch3
    is_sq = ((fl % 32) >= 16) & ch3
    S = jnp.sum(jnp.where(is_sum, p, 0.0))
    SS = jnp.sum(jnp.where(is_sq, p, 0.0))
    M = jnp.float32(3 * N)
    mu = S / M
    var = (SS - S * S / M) / (M - 1.0)
    inv_sd = lax.rsqrt(var)
    f = fac_ref[0, 0]
    vs = fac_ref[0, 1]
    s1 = 2.0 * f / vs
    g = g_ref[...]
    rows = lax.broadcasted_iota(jnp.int32, g.shape, 0)
    sig = 1.0 / (1.0 + jnp.exp(-g))
    sig4 = 1.0 / (1.0 + jnp.exp(-(g - 4.0)))
    dm = (g - mu) * inv_sd * (s1 / 6.0)
    o_ref[...] = jnp.where(
        rows < 3,
        dm,
        jnp.where(
            rows < 7,
            g,
            jnp.where(rows < 10, sig * s1, jnp.where(rows == 13, sig4, sig)),
        ),
    )


def _tc_activate(g_raw, partials, fac):
    BN = 8192
    grid = (pl.cdiv(N, BN),)
    return pl.pallas_call(
        _tc_act_body,
        grid=grid,
        in_specs=[
            pl.BlockSpec((NCH, BN), lambda i: (0, i)),
            pl.BlockSpec((8, 128), lambda i: (0, 0)),
            pl.BlockSpec((1, 2), lambda i: (0, 0)),
        ],
        out_specs=pl.BlockSpec((NCH, BN), lambda i: (0, i)),
        out_shape=jax.ShapeDtypeStruct((NCH, N), jnp.float32),
    )(g_raw, partials, fac)


def kernel(hash_table, coordinates, far, voxel_size):
    cflat = jnp.pad(coordinates, ((0, NPAD - N), (0, 0))).reshape(-1)
    t16 = _sc_transpose(hash_table.reshape(-1)).reshape(TABLE_T, 16)
    g_raw, partials = _sc_gather(t16, cflat)
    fac = jnp.stack(
        [far[0].astype(jnp.float32),
         jnp.asarray(voxel_size, jnp.float32)]
    ).reshape(1, 2)
    return _tc_activate(
        g_raw.reshape(NCH, NPAD), partials.reshape(8, 128), fac
    )


# speedup vs baseline: 1.5714x; 1.1099x over previous
import dataclasses
import functools

import jax
import jax.numpy as jnp
import numpy as np
from jax import lax
from jax.experimental import pallas as pl
from jax.experimental.pallas import tpu as pltpu
from jax.experimental.pallas import tpu_sc as plsc

N = 1_000_000
TABLE_T = 2_097_152
NCH = 15
NPAD = 1 << 20
NW = 32
PER_W = NPAD // NW

TK = 2048
T_PER_W = TABLE_T // NW
TCHUNKS = T_PER_W // TK

CG = 2048
GCHUNKS = PER_W // CG

P2 = np.int32(np.uint32(2654435761).view(np.int32))
P3 = np.int32(805459861)
MASK = np.int32(TABLE_T - 1)


def _cparams(tc_tiling):
    cp = pltpu.CompilerParams()
    fields = pltpu.CompilerParams.__dataclass_fields__
    if "needs_layout_passes" in fields:
        cp = dataclasses.replace(cp, needs_layout_passes=False)
    if "use_tc_tiling_on_sc" in fields:
        cp = dataclasses.replace(cp, use_tc_tiling_on_sc=tc_tiling)
    return cp


def _sc_transpose(tab):
    mesh = plsc.VectorSubcoreMesh(core_axis_name="c", subcore_axis_name="s")

    @functools.partial(
        pl.kernel,
        mesh=mesh,
        compiler_params=_cparams(True),
        out_type=jax.ShapeDtypeStruct((TABLE_T * 16,), jnp.float32),
        scratch_types=[
            pltpu.VMEM((16, TK), jnp.float32),
            pltpu.VMEM((16 * TK,), jnp.float32),
            pltpu.SemaphoreType.DMA,
        ],
    )
    def k(tab_hbm, t16_hbm, cbuf, obuf, sem):
        wid = lax.axis_index("s") * 2 + lax.axis_index("c")
        t_base_w = wid * T_PER_W
        lane = lax.iota(jnp.int32, 16)
        lrow = jnp.minimum(lane, 14)

        @pl.loop(0, TCHUNKS)
        def _chunk(ci):
            t0 = pl.multiple_of(t_base_w + ci * TK, TK)
            pltpu.async_copy(
                tab_hbm.at[pl.ds(0, 8), pl.ds(t0, TK)],
                cbuf.at[pl.ds(0, 8)],
                sem,
            )
            pltpu.async_copy(
                tab_hbm.at[pl.ds(8, 7), pl.ds(t0, TK)],
                cbuf.at[pl.ds(8, 7)],
                sem,
            )
            pltpu.make_async_copy(
                tab_hbm.at[pl.ds(0, 8), pl.ds(t0, TK)],
                cbuf.at[pl.ds(0, 8)],
                sem,
            ).wait()
            pltpu.make_async_copy(
                tab_hbm.at[pl.ds(8, 7), pl.ds(t0, TK)],
                cbuf.at[pl.ds(8, 7)],
                sem,
            ).wait()

            @plsc.parallel_loop(0, TK, 1, unroll=8)
            def _row(r):
                v = plsc.load_gather(cbuf, [lrow, jnp.broadcast_to(r, (16,))])
                obuf[pl.ds(r * 16, 16)] = v

            pltpu.sync_copy(obuf, t16_hbm.at[pl.ds(t0 * 16, 16 * TK)])

    return k(tab)


def _sc_gather(t16, cflat):
    mesh = plsc.VectorSubcoreMesh(core_axis_name="c", subcore_axis_name="s")

    @functools.partial(
        pl.kernel,
        mesh=mesh,
        compiler_params=_cparams(False),
        out_type=(
            jax.ShapeDtypeStruct((NCH * NPAD,), jnp.float32),
            jax.ShapeDtypeStruct((NW * 2 * 16,), jnp.float32),
        ),
        scratch_types=[
            pltpu.VMEM((3 * CG,), jnp.int32),
            pltpu.VMEM((CG,), jnp.int32),
            pltpu.VMEM((CG,), jnp.int32),
            pltpu.VMEM((CG, 16), jnp.float32),
            pltpu.VMEM((CG, 16), jnp.float32),
            pltpu.VMEM((16 * CG,), jnp.float32),
            pltpu.VMEM((16,), jnp.float32),
            pltpu.VMEM((16,), jnp.float32),
            pltpu.SemaphoreType.DMA,
            pltpu.SemaphoreType.DMA,
            pltpu.SemaphoreType.DMA,
        ],
    )
    def k(t16_hbm, cf_hbm, g_hbm, part_hbm, c3, idx0, idx1, gb0, gb1, gbx,
          acc, accsq, sg0, sg1, semw):
        wid = lax.axis_index("s") * 2 + lax.axis_index("c")
        base_w = wid * PER_W
        acc[...] = jnp.zeros((16,), jnp.float32)
        accsq[...] = jnp.zeros((16,), jnp.float32)
        idxs = (idx0, idx1)
        gbs = (gb0, gb1)
        sgs = (sg0, sg1)
        lane = lax.iota(jnp.int32, 16)
        laneCG = lane * CG

        def chunk_base(k_):
            return pl.multiple_of(base_w + k_ * CG, CG)

        def hash_chunk(k_, b):
            base = chunk_base(k_)
            idx = idxs[b]
            pltpu.sync_copy(cf_hbm.at[pl.ds(base * 3, 3 * CG)], c3)

            @plsc.parallel_loop(0, CG, 16, unroll=4)
            def _hash16(j):
                i0 = j * 3 + lane * 3
                x = plsc.load_gather(c3, [i0])
                y = plsc.load_gather(c3, [i0 + 1])
                z = plsc.load_gather(c3, [i0 + 2])
                h = x ^ (y * P2) ^ (z * P3)
                idx[pl.ds(j, 16)] = h & MASK

        def fire(b):
            pltpu.async_copy(t16_hbm.at[idxs[b]], gbs[b], sgs[b])

        def wait_gather(b):
            pltpu.make_async_copy(t16_hbm.at[idxs[b]], gbs[b], sgs[b]).wait()

        def transform(k_, b):
            base = chunk_base(k_)
            gb = gbs[b]

            def _row(r, c):
                a, asq = c
                v = gb[r, :]
                m = jnp.where(base + r < N, jnp.float32(1.0), jnp.float32(0.0))
                plsc.store_scatter(gbx, [laneCG + r], v)
                return (a + v * m, asq + v * v * m)

            a, asq = plsc.parallel_loop(
                0, CG, 1, unroll=8, carry=(acc[...], accsq[...])
            )(_row)
            acc[...] = a
            accsq[...] = asq

        def fire_writes(k_):
            base = chunk_base(k_)
            for ch in range(NCH):
                dst0 = pl.multiple_of(np.int32(ch * NPAD) + base, CG)
                pltpu.async_copy(
                    gbx.at[pl.ds(ch * CG, CG)],
                    g_hbm.at[pl.ds(dst0, CG)],
                    semw,
                )

        def wait_writes(k_):
            base = chunk_base(k_)
            for ch in range(NCH):
                dst0 = pl.multiple_of(np.int32(ch * NPAD) + base, CG)
                pltpu.make_async_copy(
                    gbx.at[pl.ds(ch * CG, CG)],
                    g_hbm.at[pl.ds(dst0, CG)],
                    semw,
                ).wait()

        hash_chunk(0, 0)
        fire(0)

        @pl.loop(0, GCHUNKS // 2)
        def _pair(kk):
            k0 = kk * 2
            hash_chunk(k0 + 1, 1)
            fire(1)
            wait_gather(0)
            transform(k0, 0)
            fire_writes(k0)

            @pl.when(k0 + 2 < GCHUNKS)
            def _():
                hash_chunk(k0 + 2, 0)
                fire(0)

            wait_gather(1)
            wait_writes(k0)
            transform(k0 + 1, 1)
            fire_writes(k0 + 1)
            wait_writes(k0 + 1)

        pltpu.sync_copy(acc, part_hbm.at[pl.ds(pl.multiple_of(wid * 32, 16), 16)])
        pltpu.sync_copy(
            accsq, part_hbm.at[pl.ds(pl.multiple_of(wid * 32 + 16, 16), 16)]
        )

    return k(t16, cflat)


def _tc_act_body(g_ref, p_ref, fac_ref, o_ref):
    p = p_ref[...]
    fl = lax.broadcasted_iota(jnp.int32, p.shape, 0) * 128 + lax.broadcasted_iota(
        jnp.int32, p.shape, 1
    )
    ch3 = (fl % 16) < 3
    is_sum = ((fl % 32) < 16) & ch3
    is_sq = ((fl % 32) >= 16) & ch3
    S = jnp.sum(jnp.where(is_sum, p, 0.0))
    SS = jnp.sum(jnp.where(is_sq, p, 0.0))
    M = jnp.float32(3 * N)
    mu = S / M
    var = (SS - S * S / M) / (M - 1.0)
    inv_sd = lax.rsqrt(var)
    f = fac_ref[0, 0]
    vs = fac_ref[0, 1]
    s1 = 2.0 * f / vs
    g = g_ref[...]
    rows = lax.broadcasted_iota(jnp.int32, g.shape, 0)
    sig = 1.0 / (1.0 + jnp.exp(-g))
    sig4 = 1.0 / (1.0 + jnp.exp(-(g - 4.0)))
    dm = (g - mu) * inv_sd * (s1 / 6.0)
    o_ref[...] = jnp.where(
        rows < 3,
        dm,
        jnp.where(
            rows < 7,
            g,
            jnp.where(rows < 10, sig * s1, jnp.where(rows == 13, sig4, sig)),
        ),
    )


def _tc_activate(g_raw, partials, fac):
    BN = 8192
    grid = (pl.cdiv(N, BN),)
    return pl.pallas_call(
        _tc_act_body,
        grid=grid,
        in_specs=[
            pl.BlockSpec((NCH, BN), lambda i: (0, i)),
            pl.BlockSpec((8, 128), lambda i: (0, 0)),
            pl.BlockSpec((1, 2), lambda i: (0, 0)),
        ],
        out_specs=pl.BlockSpec((NCH, BN), lambda i: (0, i)),
        out_shape=jax.ShapeDtypeStruct((NCH, N), jnp.float32),
    )(g_raw, partials, fac)


def kernel(hash_table, coordinates, far, voxel_size):
    cflat = jnp.pad(coordinates, ((0, NPAD - N), (0, 0))).reshape(-1)
    t16 = _sc_transpose(hash_table).reshape(TABLE_T, 16)
    g_raw, partials = _sc_gather(t16, cflat)
    fac = jnp.stack(
        [far[0].astype(jnp.float32),
         jnp.asarray(voxel_size, jnp.float32)]
    ).reshape(1, 2)
    return _tc_activate(
        g_raw.reshape(NCH, NPAD), partials.reshape(8, 128), fac
    )
